# baseline (speedup 1.0000x reference)
import jax
import jax.numpy as jnp
from jax.experimental import pallas as pl
from jax.experimental.pallas import tpu as pltpu
from jax.experimental.pallas import tpu_sc as plsc

_K = 8192
_D = 32
_N = 8192
_RB = 1024
_NB = _N // _RB
_PANEL = 2048
_NP = _K // _PANEL
_BETA = 0.25

_GATHER_WINDOW = 256


def _argmin_body(rowsum_ref, esum_ref, ze_ref, e_ref, idx_ref, loss_ref):
    i = pl.program_id(0)
    ze_b = ze_ref[...].astype(jnp.bfloat16).astype(jnp.float32)
    mm = jax.lax.dot_general(
        ze_b, e_ref[...], (((1,), (1,)), ((), ())),
        preferred_element_type=jnp.float32)
    dist = (rowsum_ref[...] + esum_ref[...]) - mm * 2.0

    pwin = jnp.zeros((_RB, 1), jnp.int32)
    val = jnp.zeros((_RB, 1), jnp.float32)
    acc = jnp.full((_RB, 1), jnp.inf, jnp.float32)
    for p in range(_NP):
        m = jnp.min(dist[:, p * _PANEL:(p + 1) * _PANEL], axis=1,
                    keepdims=True)
        upd = m < acc
        pwin = jnp.where(upd, p, pwin)
        val = jnp.where(upd, m, val)
        acc = jnp.where(upd, m.astype(jnp.bfloat16).astype(jnp.float32), acc)

    iota = jax.lax.broadcasted_iota(jnp.int32, (_RB, _K), 1)
    mask = (dist == val) & ((iota // _PANEL) == pwin)
    idx_ref[...] = jnp.min(jnp.where(mask, iota, _K), axis=1,
                           keepdims=True).astype(jnp.int32)
    part = jnp.sum(val, keepdims=True)

    @pl.when(i == 0)
    def _():
        loss_ref[...] = jnp.zeros((1, 1), jnp.float32)

    loss_ref[...] += part

    @pl.when(i == _NB - 1)
    def _():
        loss_ref[...] = loss_ref[...] * ((1.0 + _BETA) / (_N * _D))


def _nearest_code(rowsum, esum, ze, e):
    return pl.pallas_call(
        _argmin_body,
        grid=(_NB,),
        in_specs=[
            pl.BlockSpec((_RB, 1), lambda i: (i, 0)),
            pl.BlockSpec((1, _K), lambda i: (0, 0)),
            pl.BlockSpec((_RB, _D), lambda i: (i, 0)),
            pl.BlockSpec((_K, _D), lambda i: (0, 0)),
        ],
        out_specs=[
            pl.BlockSpec((_RB, 1), lambda i: (i, 0)),
            pl.BlockSpec((1, 1), lambda i: (0, 0)),
        ],
        out_shape=[
            jax.ShapeDtypeStruct((_N, 1), jnp.int32),
            jax.ShapeDtypeStruct((1, 1), jnp.float32),
        ],
    )(rowsum, esum, ze, e)


_GROW = 128


def _sc_gather(e_pad, idx_row):
    mesh = plsc.VectorSubcoreMesh(core_axis_name="core",
                                  subcore_axis_name="subcore")

    @pl.kernel(out_type=jax.ShapeDtypeStruct((_N, _GROW), e_pad.dtype),
               mesh=mesh)
    def kernel(e_hbm, i_hbm, o_hbm):
        def body(i_vmem, o_vmem):
            pltpu.sync_copy(e_hbm.at[i_vmem.at[0]], o_vmem)

        pltpu.emit_pipeline(
            body,
            grid=(_N // _GATHER_WINDOW,),
            in_specs=[pl.BlockSpec((1, _GATHER_WINDOW),
                                   index_map=lambda i: (0, i))],
            out_specs=[pl.BlockSpec((_GATHER_WINDOW, _GROW),
                                    index_map=lambda i: (i, 0))],
            core_axis_name="subcore",
            dimension_semantics=(pltpu.PARALLEL,),
        )(i_hbm, o_hbm)

    return kernel(e_pad, idx_row)


def kernel(inputs, E):
    ze_tensor = jnp.transpose(inputs, (0, 2, 3, 1))
    ze = ze_tensor.reshape(-1, _D)
    rowsum = jnp.sum(ze ** 2, axis=1, keepdims=True)
    esum = jnp.sum(E ** 2, axis=1).reshape(1, _K)
    idx, loss = _nearest_code(rowsum, esum, ze, E)
    e_pad = jnp.pad(E, ((0, 0), (0, _GROW - _D)))
    zq = _sc_gather(e_pad, idx.reshape(1, _N))[:, :_D]
    zq_tensor = zq.reshape(ze_tensor.shape)
    zq_st = ze_tensor + (zq_tensor - ze_tensor)
    return (loss.reshape(()), jnp.transpose(zq_st, (0, 3, 1, 2)))


# speedup vs baseline: 1.0608x; 1.0608x over previous
import jax
import jax.numpy as jnp
from jax.experimental import pallas as pl
from jax.experimental.pallas import tpu as pltpu
from jax.experimental.pallas import tpu_sc as plsc

_K = 8192
_D = 32
_N = 8192
_RB = 1024
_NB = _N // _RB
_PANEL = 2048
_NP = _K // _PANEL
_BETA = 0.25

_GATHER_WINDOW = 256


def _argmin_body(rowsum_ref, esum_ref, ze_ref, e_ref, idx_ref, loss_ref):
    i = pl.program_id(0)
    ze_b = ze_ref[...].astype(jnp.bfloat16).astype(jnp.float32)
    mm = jax.lax.dot_general(
        ze_b, e_ref[...], (((1,), (1,)), ((), ())),
        preferred_element_type=jnp.float32)
    dist = (rowsum_ref[...] + esum_ref[...]) - mm * 2.0

    idx = jnp.zeros((_RB, 1), jnp.int32)
    val = jnp.zeros((_RB, 1), jnp.float32)
    acc = jnp.full((_RB, 1), jnp.inf, jnp.float32)
    iota = jax.lax.broadcasted_iota(jnp.int32, (_RB, _PANEL), 1)
    for p in range(_NP):
        dp = dist[:, p * _PANEL:(p + 1) * _PANEL]
        m = jnp.min(dp, axis=1, keepdims=True)
        a = jnp.min(jnp.where(dp == m, iota, _K), axis=1,
                    keepdims=True).astype(jnp.int32) + p * _PANEL
        upd = m < acc
        idx = jnp.where(upd, a, idx)
        val = jnp.where(upd, m, val)
        acc = jnp.where(upd, m.astype(jnp.bfloat16).astype(jnp.float32), acc)

    idx_ref[...] = idx
    part = jnp.sum(val, keepdims=True)

    @pl.when(i == 0)
    def _():
        loss_ref[...] = jnp.zeros((1, 1), jnp.float32)

    loss_ref[...] += part


def _nearest_code(rowsum, esum, ze, e):
    nrows = ze.shape[0]
    return pl.pallas_call(
        _argmin_body,
        grid=(nrows // _RB,),
        in_specs=[
            pl.BlockSpec((_RB, 1), lambda i: (i, 0)),
            pl.BlockSpec((1, _K), lambda i: (0, 0)),
            pl.BlockSpec((_RB, _D), lambda i: (i, 0)),
            pl.BlockSpec((_K, _D), lambda i: (0, 0)),
        ],
        out_specs=[
            pl.BlockSpec((_RB, 1), lambda i: (i, 0)),
            pl.BlockSpec((1, 1), lambda i: (0, 0)),
        ],
        out_shape=[
            jax.ShapeDtypeStruct((nrows, 1), jnp.int32),
            jax.ShapeDtypeStruct((1, 1), jnp.float32),
        ],
    )(rowsum, esum, ze, e)


_GROW = 128


def _sc_gather(e_pad, idx_row):
    n_idx = idx_row.shape[1]
    window = min(_GATHER_WINDOW, n_idx // 32)
    mesh = plsc.VectorSubcoreMesh(core_axis_name="core",
                                  subcore_axis_name="subcore")

    @pl.kernel(out_type=jax.ShapeDtypeStruct((n_idx, _GROW), e_pad.dtype),
               mesh=mesh)
    def kernel(e_hbm, i_hbm, o_hbm):
        def body(i_vmem, o_vmem):
            pltpu.sync_copy(e_hbm.at[i_vmem.at[0]], o_vmem)

        pltpu.emit_pipeline(
            body,
            grid=(n_idx // window,),
            in_specs=[pl.BlockSpec((1, window),
                                   index_map=lambda i: (0, i))],
            out_specs=[pl.BlockSpec((window, _GROW),
                                    index_map=lambda i: (i, 0))],
            core_axis_name="subcore",
            dimension_semantics=(pltpu.PARALLEL,),
        )(i_hbm, o_hbm)

    return kernel(e_pad, idx_row)


def kernel(inputs, E):
    ze_tensor = jnp.transpose(inputs, (0, 2, 3, 1))
    ze = ze_tensor.reshape(-1, _D)
    rowsum = jnp.sum(ze ** 2, axis=1, keepdims=True)
    esum = jnp.sum(E ** 2, axis=1).reshape(1, _K)
    e_pad = jnp.pad(E, ((0, 0), (0, _GROW - _D)))
    half = _N // 2
    zq_halves, loss_sums = [], []
    for h in range(2):
        sl = slice(h * half, (h + 1) * half)
        idx_h, s_h = _nearest_code(rowsum[sl], esum, ze[sl], E)
        zq_halves.append(_sc_gather(e_pad, idx_h.reshape(1, half))[:, :_D])
        loss_sums.append(s_h)
    loss = (loss_sums[0] + loss_sums[1]) * ((1.0 + _BETA) / (_N * _D))
    zq_tensor = jnp.concatenate(zq_halves, axis=0).reshape(ze_tensor.shape)
    zq_st = ze_tensor + (zq_tensor - ze_tensor)
    return (loss.reshape(()), jnp.transpose(zq_st, (0, 3, 1, 2)))


# speedup vs baseline: 1.2363x; 1.1654x over previous
import jax
import jax.numpy as jnp
from jax.experimental import pallas as pl
from jax.experimental.pallas import tpu as pltpu
from jax.experimental.pallas import tpu_sc as plsc

_K = 8192
_D = 32
_N = 8192
_RB = 1024
_NB = _N // _RB
_PANEL = 2048
_NP = _K // _PANEL
_BETA = 0.25

_GATHER_WINDOW = 256


def _argmin_body(rowsum_ref, esum_ref, ze_ref, e_ref, idx_ref, loss_ref):
    i = pl.program_id(0)
    ze_b = ze_ref[...].astype(jnp.bfloat16).astype(jnp.float32)
    mm = jax.lax.dot_general(
        ze_b, e_ref[...], (((1,), (1,)), ((), ())),
        preferred_element_type=jnp.float32)
    dist = (rowsum_ref[...] + esum_ref[...]) - mm * 2.0

    idx = jnp.zeros((_RB, 1), jnp.int32)
    val = jnp.zeros((_RB, 1), jnp.float32)
    acc = jnp.full((_RB, 1), jnp.inf, jnp.float32)
    iota = jax.lax.broadcasted_iota(jnp.int32, (_RB, _PANEL), 1)
    for p in range(_NP):
        dp = dist[:, p * _PANEL:(p + 1) * _PANEL]
        m = jnp.min(dp, axis=1, keepdims=True)
        a = jnp.min(jnp.where(dp == m, iota, _K), axis=1,
                    keepdims=True).astype(jnp.int32) + p * _PANEL
        upd = m < acc
        idx = jnp.where(upd, a, idx)
        val = jnp.where(upd, m, val)
        acc = jnp.where(upd, m.astype(jnp.bfloat16).astype(jnp.float32), acc)

    idx_ref[...] = idx
    part = jnp.sum(val, keepdims=True)

    @pl.when(i == 0)
    def _():
        loss_ref[...] = jnp.zeros((1, 1), jnp.float32)

    loss_ref[...] += part

    @pl.when(i == _NB - 1)
    def _():
        loss_ref[...] = loss_ref[...] * ((1.0 + _BETA) / (_N * _D))


def _nearest_code(rowsum, esum, ze, e):
    return pl.pallas_call(
        _argmin_body,
        grid=(_NB,),
        in_specs=[
            pl.BlockSpec((_RB, 1), lambda i: (i, 0)),
            pl.BlockSpec((1, _K), lambda i: (0, 0)),
            pl.BlockSpec((_RB, _D), lambda i: (i, 0)),
            pl.BlockSpec((_K, _D), lambda i: (0, 0)),
        ],
        out_specs=[
            pl.BlockSpec((_RB, 1), lambda i: (i, 0)),
            pl.BlockSpec((1, 1), lambda i: (0, 0)),
        ],
        out_shape=[
            jax.ShapeDtypeStruct((_N, 1), jnp.int32),
            jax.ShapeDtypeStruct((1, 1), jnp.float32),
        ],
    )(rowsum, esum, ze, e)


_GROW = 128


def _sc_gather(e_pad, idx_row):
    mesh = plsc.VectorSubcoreMesh(core_axis_name="core",
                                  subcore_axis_name="subcore")

    @pl.kernel(out_type=jax.ShapeDtypeStruct((_N, _GROW), e_pad.dtype),
               mesh=mesh)
    def kernel(e_hbm, i_hbm, o_hbm):
        def body(i_vmem, o_vmem):
            pltpu.sync_copy(e_hbm.at[i_vmem.at[0]], o_vmem)

        pltpu.emit_pipeline(
            body,
            grid=(_N // _GATHER_WINDOW,),
            in_specs=[pl.BlockSpec((1, _GATHER_WINDOW),
                                   index_map=lambda i: (0, i))],
            out_specs=[pl.BlockSpec((_GATHER_WINDOW, _GROW),
                                    index_map=lambda i: (i, 0))],
            core_axis_name="subcore",
            dimension_semantics=(pltpu.PARALLEL,),
        )(i_hbm, o_hbm)

    return kernel(e_pad, idx_row)


def kernel(inputs, E):
    ze_tensor = jnp.transpose(inputs, (0, 2, 3, 1))
    ze = ze_tensor.reshape(-1, _D)
    rowsum = jnp.sum(ze ** 2, axis=1, keepdims=True)
    esum = jnp.sum(E ** 2, axis=1).reshape(1, _K)
    idx, loss = _nearest_code(rowsum, esum, ze, E)
    e_pad = jnp.pad(E, ((0, 0), (0, _GROW - _D)))
    zq = _sc_gather(e_pad, idx.reshape(1, _N))[:, :_D]
    zq_tensor = zq.reshape(ze_tensor.shape)
    zq_st = ze_tensor + (zq_tensor - ze_tensor)
    return (loss.reshape(()), jnp.transpose(zq_st, (0, 3, 1, 2)))
